# baseline (device time: 103208 ns/iter reference)
import jax
import jax.numpy as jnp
from jax import lax
from jax.experimental import pallas as pl
from jax.experimental.pallas import tpu as pltpu

N_DEV = 16
BLK = 64


def kernel(x, Wq, K_ext, V_ext, Wo):
    B, Sq, F = x.shape
    _, Skv_sh, Hq, Dh = K_ext.shape
    HD = Hq * Dh
    Fo = Wo.shape[1]
    ROWS = Sq + Hq

    def body(x_ref, wq_ref, k_ref, v_ref, wo_ref, out_ref,
             acc_ref, comm_ref, send_sems, recv_sems):
        my = lax.axis_index("i")

        qb = lax.broadcasted_iota(jnp.int32, (Sq, Skv_sh), 0) // BLK
        kb = my * (Skv_sh // BLK) + (
            lax.broadcasted_iota(jnp.int32, (Sq, Skv_sh), 1) // BLK
        )
        mask = (qb == kb) | (kb == 0) | ((qb + kb) % 3 == 0)

        for b in range(B):
            Qb = jnp.dot(x_ref[b], wq_ref[...],
                         preferred_element_type=jnp.float32)
            for h in range(Hq):
                q_bh = Qb[:, h * Dh:(h + 1) * Dh]
                k_bh = k_ref[b, :, h, :]
                s = lax.dot_general(
                    q_bh, k_bh, (((1,), (1,)), ((), ())),
                    preferred_element_type=jnp.float32) * 0.125
                w = jnp.where(mask, jnp.exp(s), 0.0)
                acc_ref[b, Sq + h, :] = jnp.sum(w, axis=1)
                acc_ref[b, :Sq, h * Dh:(h + 1) * Dh] = jnp.dot(
                    w, v_ref[b, :, h, :],
                    preferred_element_type=jnp.float32)

        send_rdmas = []
        for p in range(N_DEV):
            rdma = pltpu.make_async_remote_copy(
                src_ref=acc_ref,
                dst_ref=comm_ref.at[my],
                send_sem=send_sems.at[p],
                recv_sem=recv_sems.at[my],
                device_id=(p,),
                device_id_type=pl.DeviceIdType.MESH,
            )
            send_rdmas.append(rdma)

            @pl.when(p != my)
            def _():
                rdma.start()

        for s_ in range(N_DEV):
            recv = pltpu.make_async_remote_copy(
                src_ref=comm_ref.at[s_],
                dst_ref=comm_ref.at[s_],
                send_sem=send_sems.at[s_],
                recv_sem=recv_sems.at[s_],
                device_id=(0,),
                device_id_type=pl.DeviceIdType.MESH,
            )

            @pl.when(s_ != my)
            def _():
                recv.wait_recv()

        total = acc_ref[...]
        for s_ in range(N_DEV):
            total = total + jnp.where(s_ == my, 0.0, comm_ref[s_])

        for p in range(N_DEV):
            @pl.when(p != my)
            def _():
                send_rdmas[p].wait_send()

        for b in range(B):
            ctx = total[b, :Sq, :]
            lsum = total[b, Sq:Sq + Hq, :]
            inv_t = (1.0 / lsum).T
            scale = jnp.broadcast_to(
                inv_t[:, :, None], (Sq, Hq, Dh)).reshape(Sq, HD)
            out_ref[b] = jnp.dot(ctx * scale, wo_ref[...],
                                 preferred_element_type=jnp.float32)

    return pl.pallas_call(
        body,
        out_shape=jax.ShapeDtypeStruct((B, Sq, Fo), jnp.float32),
        in_specs=[pl.BlockSpec(memory_space=pltpu.VMEM)] * 5,
        out_specs=pl.BlockSpec(memory_space=pltpu.VMEM),
        scratch_shapes=[
            pltpu.VMEM((B, ROWS, HD), jnp.float32),
            pltpu.VMEM((N_DEV, B, ROWS, HD), jnp.float32),
            pltpu.SemaphoreType.DMA((N_DEV,)),
            pltpu.SemaphoreType.DMA((N_DEV,)),
        ],
    )(x, Wq, K_ext, V_ext, Wo)


# device time: 56032 ns/iter; 1.8419x vs baseline; 1.8419x over previous
import jax
import jax.numpy as jnp
from jax import lax
from jax.experimental import pallas as pl
from jax.experimental.pallas import tpu as pltpu

N_DEV = 16
BLK = 64


def kernel(x, Wq, K_ext, V_ext, Wo):
    B, Sq, F = x.shape
    _, Skv_sh, Hq, Dh = K_ext.shape
    HD = Hq * Dh
    Fo = Wo.shape[1]
    ROWS = Sq + Hq

    def body(x_ref, wq_ref, k_ref, v_ref, wo_ref, out_ref,
             acc_ref, comm_ref, send_sems, recv_sems):
        my = lax.axis_index("i")

        qb = lax.broadcasted_iota(jnp.int32, (Sq, Skv_sh), 0) // BLK
        kb = my * (Skv_sh // BLK) + (
            lax.broadcasted_iota(jnp.int32, (Sq, Skv_sh), 1) // BLK
        )
        mask = (qb == kb) | (kb == 0) | ((qb + kb) % 3 == 0)

        for b in range(B):
            Qb = jnp.dot(x_ref[b], wq_ref[...],
                         preferred_element_type=jnp.float32)
            for h in range(Hq):
                q_bh = Qb[:, h * Dh:(h + 1) * Dh]
                k_bh = k_ref[b, :, h, :]
                s = lax.dot_general(
                    q_bh, k_bh, (((1,), (1,)), ((), ())),
                    preferred_element_type=jnp.float32) * 0.125
                w = jnp.where(mask, jnp.exp(s), 0.0)
                acc_ref[b, Sq + h, :] = jnp.sum(w, axis=1)
                acc_ref[b, :Sq, h * Dh:(h + 1) * Dh] = jnp.dot(
                    w, v_ref[b, :, h, :],
                    preferred_element_type=jnp.float32)

        for ph, c in enumerate((1, 3, 4, 8)):
            partner = jnp.bitwise_xor(my, c)
            rdma = pltpu.make_async_remote_copy(
                src_ref=acc_ref,
                dst_ref=comm_ref.at[ph],
                send_sem=send_sems.at[ph],
                recv_sem=recv_sems.at[ph],
                device_id=(partner,),
                device_id_type=pl.DeviceIdType.MESH,
            )
            rdma.start()
            rdma.wait()
            acc_ref[...] = acc_ref[...] + comm_ref[ph]

        total = acc_ref[...]

        for b in range(B):
            ctx = total[b, :Sq, :]
            lsum = total[b, Sq:Sq + Hq, :]
            inv_t = (1.0 / lsum).T
            scale = jnp.broadcast_to(
                inv_t[:, :, None], (Sq, Hq, Dh)).reshape(Sq, HD)
            out_ref[b] = jnp.dot(ctx * scale, wo_ref[...],
                                 preferred_element_type=jnp.float32)

    return pl.pallas_call(
        body,
        out_shape=jax.ShapeDtypeStruct((B, Sq, Fo), jnp.float32),
        in_specs=[pl.BlockSpec(memory_space=pltpu.VMEM)] * 5,
        out_specs=pl.BlockSpec(memory_space=pltpu.VMEM),
        scratch_shapes=[
            pltpu.VMEM((B, ROWS, HD), jnp.float32),
            pltpu.VMEM((4, B, ROWS, HD), jnp.float32),
            pltpu.SemaphoreType.DMA((4,)),
            pltpu.SemaphoreType.DMA((4,)),
        ],
    )(x, Wq, K_ext, V_ext, Wo)


# device time: 42042 ns/iter; 2.4549x vs baseline; 1.3328x over previous
import jax
import jax.numpy as jnp
from jax import lax
from jax.experimental import pallas as pl
from jax.experimental.pallas import tpu as pltpu

N_DEV = 16
BLK = 64


def kernel(x, Wq, K_ext, V_ext, Wo):
    B, Sq, F = x.shape
    _, Skv_sh, Hq, Dh = K_ext.shape
    HD = Hq * Dh
    Fo = Wo.shape[1]
    ROWS = Sq + Hq

    def body(x_ref, wq_ref, k_ref, v_ref, wo_ref, out_ref,
             acc_ref, sbuf_ref, comm_ref, send_sems, recv_sems):
        my = lax.axis_index("i")

        qb = lax.broadcasted_iota(jnp.int32, (Sq, Skv_sh), 0) // BLK
        kb = my * (Skv_sh // BLK) + (
            lax.broadcasted_iota(jnp.int32, (Sq, Skv_sh), 1) // BLK
        )
        mask = (qb == kb) | (kb == 0) | ((qb + kb) % 3 == 0)

        for b in range(B):
            Qb = jnp.dot(x_ref[b], wq_ref[...],
                         preferred_element_type=jnp.float32)
            for h in range(Hq):
                q_bh = Qb[:, h * Dh:(h + 1) * Dh]
                k_bh = k_ref[b, :, h, :]
                s = lax.dot_general(
                    q_bh, k_bh, (((1,), (1,)), ((), ())),
                    preferred_element_type=jnp.float32) * 0.125
                w = jnp.where(mask, jnp.exp(s), 0.0)
                acc_ref[b, Sq + h, :] = jnp.sum(w, axis=1)
                acc_ref[b, :Sq, h * Dh:(h + 1) * Dh] = jnp.dot(
                    w, v_ref[b, :, h, :],
                    preferred_element_type=jnp.float32)

        for ph, c in enumerate((1, 3, 4, 8)):
            partner = jnp.bitwise_xor(my, c)
            sbuf_ref[...] = acc_ref[...].astype(jnp.bfloat16)
            rdma = pltpu.make_async_remote_copy(
                src_ref=sbuf_ref,
                dst_ref=comm_ref.at[ph],
                send_sem=send_sems.at[ph],
                recv_sem=recv_sems.at[ph],
                device_id=(partner,),
                device_id_type=pl.DeviceIdType.MESH,
            )
            rdma.start()
            rdma.wait()
            acc_ref[...] = acc_ref[...] + comm_ref[ph].astype(jnp.float32)

        total = acc_ref[...]

        for b in range(B):
            ctx = total[b, :Sq, :]
            lsum = total[b, Sq:Sq + Hq, :]
            inv_t = (1.0 / lsum).T
            scale = jnp.broadcast_to(
                inv_t[:, :, None], (Sq, Hq, Dh)).reshape(Sq, HD)
            out_ref[b] = jnp.dot(ctx * scale, wo_ref[...],
                                 preferred_element_type=jnp.float32)

    return pl.pallas_call(
        body,
        out_shape=jax.ShapeDtypeStruct((B, Sq, Fo), jnp.float32),
        in_specs=[pl.BlockSpec(memory_space=pltpu.VMEM)] * 5,
        out_specs=pl.BlockSpec(memory_space=pltpu.VMEM),
        scratch_shapes=[
            pltpu.VMEM((B, ROWS, HD), jnp.float32),
            pltpu.VMEM((B, ROWS, HD), jnp.bfloat16),
            pltpu.VMEM((4, B, ROWS, HD), jnp.bfloat16),
            pltpu.SemaphoreType.DMA((4,)),
            pltpu.SemaphoreType.DMA((4,)),
        ],
    )(x, Wq, K_ext, V_ext, Wo)


# device time: 33170 ns/iter; 3.1115x vs baseline; 1.2675x over previous
import jax
import jax.numpy as jnp
from jax import lax
from jax.experimental import pallas as pl
from jax.experimental.pallas import tpu as pltpu

N_DEV = 16
BLK = 64

SCHED = ((1, 3, 4, 8), (8, 4, 3, 1))


def kernel(x, Wq, K_ext, V_ext, Wo):
    B, Sq, F = x.shape
    _, Skv_sh, Hq, Dh = K_ext.shape
    HD = Hq * Dh
    Fo = Wo.shape[1]
    ROWS = Sq + Hq

    def body(x_ref, wq_ref, k_ref, v_ref, wo_ref, out_ref,
             acc_ref, sbuf_ref, comm_ref, send_sems, recv_sems):
        my = lax.axis_index("i")

        qb = lax.broadcasted_iota(jnp.int32, (Sq, Skv_sh), 0) // BLK
        kb = my * (Skv_sh // BLK) + (
            lax.broadcasted_iota(jnp.int32, (Sq, Skv_sh), 1) // BLK
        )
        mask = (qb == kb) | (kb == 0) | ((qb + kb) % 3 == 0)

        def compute_partial(b):
            Qb = jnp.dot(x_ref[b], wq_ref[...],
                         preferred_element_type=jnp.float32)
            for h in range(Hq):
                q_bh = Qb[:, h * Dh:(h + 1) * Dh]
                k_bh = k_ref[b, :, h, :]
                s = lax.dot_general(
                    q_bh, k_bh, (((1,), (1,)), ((), ())),
                    preferred_element_type=jnp.float32) * 0.125
                w = jnp.where(mask, jnp.exp(s), 0.0)
                acc_ref[b, Sq + h, :] = jnp.sum(w, axis=1)
                acc_ref[b, :Sq, h * Dh:(h + 1) * Dh] = jnp.dot(
                    w, v_ref[b, :, h, :],
                    preferred_element_type=jnp.float32)

        def xchg_start(half, ph):
            partner = jnp.bitwise_xor(my, SCHED[half][ph])
            sbuf_ref[half] = acc_ref[half].astype(jnp.bfloat16)
            rdma = pltpu.make_async_remote_copy(
                src_ref=sbuf_ref.at[half],
                dst_ref=comm_ref.at[ph, half],
                send_sem=send_sems.at[ph, half],
                recv_sem=recv_sems.at[ph, half],
                device_id=(partner,),
                device_id_type=pl.DeviceIdType.MESH,
            )
            rdma.start()
            return rdma

        def finalize(b):
            ctx = acc_ref[b, :Sq, :]
            lsum = acc_ref[b, Sq:Sq + Hq, :]
            inv_t = (1.0 / lsum).T
            scale = jnp.broadcast_to(
                inv_t[:, :, None], (Sq, Hq, Dh)).reshape(Sq, HD)
            out_ref[b] = jnp.dot(ctx * scale, wo_ref[...],
                                 preferred_element_type=jnp.float32)

        compute_partial(0)
        rdmas = [None, None]
        rdmas[0] = xchg_start(0, 0)
        compute_partial(1)
        rdmas[1] = xchg_start(1, 0)

        for ph in range(4):
            for half in range(2):
                rdmas[half].wait()
                acc_ref[half] = (acc_ref[half]
                                 + comm_ref[ph, half].astype(jnp.float32))
                if ph < 3:
                    rdmas[half] = xchg_start(half, ph + 1)
                elif half == 0:
                    finalize(0)
        finalize(1)

    return pl.pallas_call(
        body,
        out_shape=jax.ShapeDtypeStruct((B, Sq, Fo), jnp.float32),
        in_specs=[pl.BlockSpec(memory_space=pltpu.VMEM)] * 5,
        out_specs=pl.BlockSpec(memory_space=pltpu.VMEM),
        scratch_shapes=[
            pltpu.VMEM((B, ROWS, HD), jnp.float32),
            pltpu.VMEM((B, ROWS, HD), jnp.bfloat16),
            pltpu.VMEM((4, B, ROWS, HD), jnp.bfloat16),
            pltpu.SemaphoreType.DMA((4, B)),
            pltpu.SemaphoreType.DMA((4, B)),
        ],
    )(x, Wq, K_ext, V_ext, Wo)


# device time: 25143 ns/iter; 4.1048x vs baseline; 1.3193x over previous
import jax
import jax.numpy as jnp
from jax import lax
from jax.experimental import pallas as pl
from jax.experimental.pallas import tpu as pltpu

N_DEV = 16
BLK = 64

SCHED = (
    (1, 3, 4, 8),
    (3, 1, 8, 4),
    (4, 8, 1, 3),
    (8, 4, 3, 1),
)


def kernel(x, Wq, K_ext, V_ext, Wo):
    B, Sq, F = x.shape
    _, Skv_sh, Hq, Dh = K_ext.shape
    HD = Hq * Dh
    Fo = Wo.shape[1]
    ROWS = -(-(Sq + Hq) // 16) * 16
    HROWS = ROWS // 2

    def body(x_ref, wq_ref, k_ref, v_ref, wo_ref, out_ref,
             acc_ref, sbuf_ref, comm_ref, send_sems, recv_sems):
        my = lax.axis_index("i")

        barrier = pltpu.get_barrier_semaphore()
        for c in (1, 3, 4, 8):
            pl.semaphore_signal(
                barrier, inc=1,
                device_id=(jnp.bitwise_xor(my, c),),
                device_id_type=pl.DeviceIdType.MESH,
            )

        qb = lax.broadcasted_iota(jnp.int32, (Sq, Skv_sh), 0) // BLK
        kb = my * (Skv_sh // BLK) + (
            lax.broadcasted_iota(jnp.int32, (Sq, Skv_sh), 1) // BLK
        )
        mask = (qb == kb) | (kb == 0) | ((qb + kb) % 3 == 0)

        def compute_partial(b):
            acc_ref[b, Sq + Hq:ROWS, :] = jnp.zeros(
                (ROWS - Sq - Hq, HD), jnp.float32)
            Qb = jnp.dot(x_ref[b], wq_ref[...],
                         preferred_element_type=jnp.float32)
            for h in range(Hq):
                q_bh = Qb[:, h * Dh:(h + 1) * Dh]
                k_bh = k_ref[b, :, h, :]
                s = lax.dot_general(
                    q_bh, k_bh, (((1,), (1,)), ((), ())),
                    preferred_element_type=jnp.float32) * 0.125
                w = jnp.where(mask, jnp.exp(s), 0.0)
                acc_ref[b, Sq + h, :] = jnp.sum(w, axis=1)
                acc_ref[b, :Sq, h * Dh:(h + 1) * Dh] = jnp.dot(
                    w, v_ref[b, :, h, :],
                    preferred_element_type=jnp.float32)

        def stream_slice(s):
            b, lo = s // 2, (s % 2) * HROWS
            return b, lo

        def xchg_start(s, t):
            b, lo = stream_slice(s)
            partner = jnp.bitwise_xor(my, SCHED[s][t])
            sbuf_ref[b, lo:lo + HROWS, :] = (
                acc_ref[b, lo:lo + HROWS, :].astype(jnp.bfloat16))
            rdma = pltpu.make_async_remote_copy(
                src_ref=sbuf_ref.at[b, lo:lo + HROWS, :],
                dst_ref=comm_ref.at[t, s],
                send_sem=send_sems.at[t, s],
                recv_sem=recv_sems.at[t, s],
                device_id=(partner,),
                device_id_type=pl.DeviceIdType.MESH,
            )
            rdma.start()
            return rdma

        def finalize(b):
            ctx = acc_ref[b, :Sq, :]
            lsum = acc_ref[b, Sq:Sq + Hq, :]
            inv_t = (1.0 / lsum).T
            scale = jnp.broadcast_to(
                inv_t[:, :, None], (Sq, Hq, Dh)).reshape(Sq, HD)
            out_ref[b] = jnp.dot(ctx * scale, wo_ref[...],
                                 preferred_element_type=jnp.float32)

        compute_partial(0)
        pl.semaphore_wait(barrier, 4)
        rdmas = [None] * 4
        rdmas[0] = xchg_start(0, 0)
        rdmas[1] = xchg_start(1, 0)
        compute_partial(1)
        rdmas[2] = xchg_start(2, 0)
        rdmas[3] = xchg_start(3, 0)

        for t in range(4):
            for s in range(4):
                b, lo = stream_slice(s)
                rdmas[s].wait()
                acc_ref[b, lo:lo + HROWS, :] = (
                    acc_ref[b, lo:lo + HROWS, :]
                    + comm_ref[t, s].astype(jnp.float32))
                if t < 3:
                    rdmas[s] = xchg_start(s, t + 1)
                elif s == 1:
                    finalize(0)
        finalize(1)

    return pl.pallas_call(
        body,
        out_shape=jax.ShapeDtypeStruct((B, Sq, Fo), jnp.float32),
        in_specs=[pl.BlockSpec(memory_space=pltpu.VMEM)] * 5,
        out_specs=pl.BlockSpec(memory_space=pltpu.VMEM),
        scratch_shapes=[
            pltpu.VMEM((B, ROWS, HD), jnp.float32),
            pltpu.VMEM((B, ROWS, HD), jnp.bfloat16),
            pltpu.VMEM((4, 4, HROWS, HD), jnp.bfloat16),
            pltpu.SemaphoreType.DMA((4, 4)),
            pltpu.SemaphoreType.DMA((4, 4)),
        ],
        compiler_params=pltpu.CompilerParams(collective_id=0),
    )(x, Wq, K_ext, V_ext, Wo)
